# baseline (speedup 1.0000x reference)
import jax
import jax.numpy as jnp
from jax import lax
from jax.experimental import pallas as pl
from jax.experimental.pallas import tpu as pltpu
from jax.experimental.pallas import tpu_sc as plsc

N_NODES = 10000
N_EDGES = 160000
F_IN = 256
F_OUT = 256
HALF = 128

NC = 2
NS = 16

ECHK = 96
CPS = 108
EPS = CPS * ECHK
E_PAD = NS * EPS
TRI = CPS // 3
ROWS_T = 624
TAIL = N_NODES - NS * ROWS_T


def _mm_body(x_ref, w_ref, o_ref):
    o_ref[0] = jnp.dot(x_ref[...], w_ref[...],
                       preferred_element_type=jnp.float32)


def _matmul_halves(features, weight):
    MB = 1000
    grid = (NC, N_NODES // MB)
    return pl.pallas_call(
        _mm_body,
        grid=grid,
        in_specs=[
            pl.BlockSpec((MB, F_IN), lambda h, i: (i, 0)),
            pl.BlockSpec((F_IN, HALF), lambda h, i: (0, h)),
        ],
        out_specs=pl.BlockSpec((1, MB, HALF), lambda h, i: (h, i, 0)),
        out_shape=jax.ShapeDtypeStruct((NC, N_NODES, HALF), jnp.float32),
    )(features, weight)


def _sc_body(sup_hbm, src_hbm, dst_hbm, ew_hbm, out_hbm,
             srcb0, srcb1, srcb2, dstb0, dstb1, dstb2, ewb0, ewb1, ewb2,
             rows0, rows1, rows2, acc_sh,
             gsem0, gsem1, gsem2, ssem0, ssem1, ssem2,
             isem0, isem1, isem2, dsem0, dsem1, dsem2):
    c = lax.axis_index("c")
    s = lax.axis_index("s")
    srcb = (srcb0, srcb1, srcb2)
    dstb = (dstb0, dstb1, dstb2)
    ewb = (ewb0, ewb1, ewb2)
    rows = (rows0, rows1, rows2)
    gsem = (gsem0, gsem1, gsem2)
    ssem = (ssem0, ssem1, ssem2)
    isem = (isem0, isem1, isem2)
    dsem = (dsem0, dsem1, dsem2)

    def base(k):
        return (s * CPS + k) * ECHK

    def load_src_ew(k, b):
        pltpu.async_copy(src_hbm.at[pl.ds(base(k), ECHK)], srcb[b], isem[b])
        pltpu.async_copy(ew_hbm.at[pl.ds(base(k), ECHK)],
                         ewb[b].at[pl.ds(0, ECHK)], isem[b])

    def wait_src_ew(b):
        pltpu.make_async_copy(src_hbm.at[pl.ds(0, ECHK)], srcb[b],
                              isem[b]).wait()
        pltpu.make_async_copy(ew_hbm.at[pl.ds(0, ECHK)],
                              ewb[b].at[pl.ds(0, ECHK)], isem[b]).wait()

    def load_dst(k, b):
        pltpu.async_copy(dst_hbm.at[pl.ds(base(k), ECHK)], dstb[b], dsem[b])

    def wait_dst(b):
        pltpu.make_async_copy(dst_hbm.at[pl.ds(0, ECHK)], dstb[b],
                              dsem[b]).wait()

    def start_gather(b):
        pltpu.async_copy(sup_hbm.at[c].at[srcb[b]], rows[b], gsem[b])

    def wait_gather(b):
        pltpu.make_async_copy(sup_hbm.at[c, pl.ds(0, ECHK)], rows[b],
                              gsem[b]).wait()

    def start_scatter(b):
        pltpu.async_copy(rows[b], acc_sh.at[dstb[b]], ssem[b], add=True)

    def wait_scatter(b):
        pltpu.make_async_copy(sup_hbm.at[c, pl.ds(0, ECHK)], rows[b],
                              ssem[b]).wait()

    def scale(b):
        buf = rows[b]
        wref = ewb[b]

        def edge(e):
            w = wref[pl.ds(e, 16)][0]
            for j in range(HALF // 16):
                buf[e, pl.ds(j * 16, 16)] = buf[e, pl.ds(j * 16, 16)] * w
        plsc.parallel_loop(0, ECHK, unroll=4)(edge)

    def zero_row(r, _):
        for j in range(HALF // 16):
            rows0[r, pl.ds(j * 16, 16)] = jnp.zeros((16,), jnp.float32)
        return 0
    lax.fori_loop(0, ECHK, zero_row, 0)
    for k in range(6):
        pltpu.sync_copy(rows0, acc_sh.at[pl.ds(s * ROWS_T + k * ECHK, ECHK)])
    pltpu.sync_copy(rows0.at[pl.ds(0, ROWS_T - 6 * ECHK)],
                    acc_sh.at[pl.ds(s * ROWS_T + 6 * ECHK, ROWS_T - 6 * ECHK)])

    @pl.when(s == NS - 1)
    def _():
        pltpu.sync_copy(rows0.at[pl.ds(0, TAIL)],
                        acc_sh.at[pl.ds(NS * ROWS_T, TAIL)])
    plsc.subcore_barrier()

    for b in range(3):
        load_src_ew(b, b)
        load_dst(b, b)
    for b in range(3):
        wait_src_ew(b)
        start_gather(b)

    def pipe_body(t, _):
        k0 = 3 * t
        not_last = t < TRI - 1

        def process(i):
            wait_gather(i)
            scale(i)

            @pl.when(not_last)
            def _():
                load_src_ew(k0 + i + 3, i)
            wait_dst(i)
            start_scatter(i)

        def recycle(i):
            wait_scatter(i)

            @pl.when(not_last)
            def _():
                load_dst(k0 + i + 3, i)
                wait_src_ew(i)
                start_gather(i)

        process(0)
        process(1)
        recycle(0)
        process(2)
        recycle(1)
        recycle(2)
        return 0
    lax.fori_loop(0, TRI, pipe_body, 0)
    plsc.subcore_barrier()

    def relu_rows(buf, nrows):
        def relu_row(r, _):
            for j in range(HALF // 16):
                v = buf[r, pl.ds(j * 16, 16)]
                buf[r, pl.ds(j * 16, 16)] = jnp.maximum(v, 0.0)
            return 0
        lax.fori_loop(0, nrows, relu_row, 0)

    def wb_chunk(r0, nrows):
        pltpu.sync_copy(acc_sh.at[pl.ds(r0, nrows)],
                        rows0.at[pl.ds(0, nrows)])
        relu_rows(rows0, nrows)
        pltpu.sync_copy(rows0.at[pl.ds(0, nrows)],
                        out_hbm.at[pl.ds(r0, nrows), pl.ds(c * HALF, HALF)])

    for k in range(6):
        wb_chunk(s * ROWS_T + k * ECHK, ECHK)
    wb_chunk(s * ROWS_T + 6 * ECHK, ROWS_T - 6 * ECHK)

    @pl.when(s == NS - 1)
    def _():
        wb_chunk(NS * ROWS_T, TAIL)


@jax.jit
def _gnn(features, src, dst, ew, weight):
    sup = _matmul_halves(features, weight)
    mesh = plsc.VectorSubcoreMesh(core_axis_name="c", subcore_axis_name="s",
                                  num_cores=NC, num_subcores=NS)
    agg = pl.kernel(
        _sc_body,
        out_type=jax.ShapeDtypeStruct((N_NODES, F_OUT), jnp.float32),
        mesh=mesh,
        scratch_types=(
            [pltpu.VMEM((ECHK,), jnp.int32) for _ in range(3)] +
            [pltpu.VMEM((ECHK,), jnp.int32) for _ in range(3)] +
            [pltpu.VMEM((ECHK + 16,), jnp.float32) for _ in range(3)] +
            [pltpu.VMEM((ECHK, HALF), jnp.float32) for _ in range(3)] +
            [pltpu.VMEM_SHARED((N_NODES, HALF), jnp.float32)] +
            [pltpu.SemaphoreType.DMA for _ in range(12)]
        ),
    )
    return agg(sup, src, dst, ew)


def kernel(features, edge_index, edge_weight, weight):
    pad = E_PAD - N_EDGES
    src = jnp.pad(edge_index[1].astype(jnp.int32), (0, pad))
    dst = jnp.pad(edge_index[0].astype(jnp.int32), (0, pad))
    ew = jnp.pad(edge_weight, (0, pad))
    return _gnn(features, src, dst, ew, weight)


# speedup vs baseline: 1.2242x; 1.2242x over previous
import jax
import jax.numpy as jnp
from jax import lax
from jax.experimental import pallas as pl
from jax.experimental.pallas import tpu as pltpu
from jax.experimental.pallas import tpu_sc as plsc

N_NODES = 10000
N_EDGES = 160000
F_IN = 256
F_OUT = 256
HALF = 128

NC = 2
NS = 16

E_PAD = 163840
EPS = E_PAD // NS
ECHK = 160
CPS = EPS // ECHK
ROWS_T = 624
TAIL = N_NODES - NS * ROWS_T


def _mm_body(x_ref, w_ref, o_ref):
    o_ref[0] = jnp.dot(x_ref[...], w_ref[...],
                       preferred_element_type=jnp.float32)


def _matmul_halves(features, weight):
    MB = 1000
    grid = (NC, N_NODES // MB)
    return pl.pallas_call(
        _mm_body,
        grid=grid,
        in_specs=[
            pl.BlockSpec((MB, F_IN), lambda h, i: (i, 0)),
            pl.BlockSpec((F_IN, HALF), lambda h, i: (0, h)),
        ],
        out_specs=pl.BlockSpec((1, MB, HALF), lambda h, i: (h, i, 0)),
        out_shape=jax.ShapeDtypeStruct((NC, N_NODES, HALF), jnp.float32),
    )(features, weight)


def _sc_body(sup_hbm, src_hbm, dst_hbm, ew_hbm, out_hbm,
             srcb0, srcb1, dstb0, dstb1, ewb0, ewb1, rows0, rows1, acc_sh,
             gsem0, gsem1, ssem0, ssem1, isem0, isem1, dsem0, dsem1):
    c = lax.axis_index("c")
    s = lax.axis_index("s")
    srcb = (srcb0, srcb1)
    dstb = (dstb0, dstb1)
    ewb = (ewb0, ewb1)
    rows = (rows0, rows1)
    gsem = (gsem0, gsem1)
    ssem = (ssem0, ssem1)
    isem = (isem0, isem1)
    dsem = (dsem0, dsem1)

    def base(k):
        return (s * CPS + k) * ECHK

    def load_src_ew(k, b):
        pltpu.async_copy(src_hbm.at[pl.ds(base(k), ECHK)], srcb[b], isem[b])
        pltpu.async_copy(ew_hbm.at[pl.ds(base(k), ECHK)],
                         ewb[b].at[pl.ds(0, ECHK)], isem[b])

    def wait_src_ew(b):
        pltpu.make_async_copy(src_hbm.at[pl.ds(0, ECHK)], srcb[b],
                              isem[b]).wait()
        pltpu.make_async_copy(ew_hbm.at[pl.ds(0, ECHK)],
                              ewb[b].at[pl.ds(0, ECHK)], isem[b]).wait()

    def load_dst(k, b):
        pltpu.async_copy(dst_hbm.at[pl.ds(base(k), ECHK)], dstb[b], dsem[b])

    def wait_dst(b):
        pltpu.make_async_copy(dst_hbm.at[pl.ds(0, ECHK)], dstb[b],
                              dsem[b]).wait()

    def start_gather(b):
        pltpu.async_copy(sup_hbm.at[c].at[srcb[b]], rows[b], gsem[b])

    def wait_gather(b):
        pltpu.make_async_copy(sup_hbm.at[c, pl.ds(0, ECHK)], rows[b],
                              gsem[b]).wait()

    def start_scatter(b):
        pltpu.async_copy(rows[b], acc_sh.at[dstb[b]], ssem[b], add=True)

    def wait_scatter(b):
        pltpu.make_async_copy(sup_hbm.at[c, pl.ds(0, ECHK)], rows[b],
                              ssem[b]).wait()

    def scale(b):
        buf = rows[b]
        wref = ewb[b]

        def edge(e):
            w = wref[pl.ds(e, 16)][0]
            for j in range(HALF // 16):
                buf[e, pl.ds(j * 16, 16)] = buf[e, pl.ds(j * 16, 16)] * w
        plsc.parallel_loop(0, ECHK, unroll=4)(edge)

    def zero_row(r, _):
        for j in range(HALF // 16):
            rows0[r, pl.ds(j * 16, 16)] = jnp.zeros((16,), jnp.float32)
        return 0
    lax.fori_loop(0, ECHK, zero_row, 0)
    NFULL = ROWS_T // ECHK
    for k in range(NFULL):
        pltpu.sync_copy(rows0, acc_sh.at[pl.ds(s * ROWS_T + k * ECHK, ECHK)])
    pltpu.sync_copy(rows0.at[pl.ds(0, ROWS_T - NFULL * ECHK)],
                    acc_sh.at[pl.ds(s * ROWS_T + NFULL * ECHK,
                                    ROWS_T - NFULL * ECHK)])

    @pl.when(s == NS - 1)
    def _():
        pltpu.sync_copy(rows0.at[pl.ds(0, TAIL)],
                        acc_sh.at[pl.ds(NS * ROWS_T, TAIL)])
    plsc.subcore_barrier()

    load_src_ew(0, 0)
    load_src_ew(1, 1)
    load_dst(0, 0)
    load_dst(1, 1)
    wait_src_ew(0)
    start_gather(0)

    def pipe_body(k2, _):
        ka = 2 * k2
        not_last = k2 < CPS // 2 - 1
        wait_src_ew(1)
        start_gather(1)
        wait_gather(0)
        scale(0)

        @pl.when(not_last)
        def _():
            load_src_ew(ka + 2, 0)
        wait_dst(0)
        start_scatter(0)
        wait_gather(1)
        scale(1)

        @pl.when(not_last)
        def _():
            load_src_ew(ka + 3, 1)
        wait_dst(1)
        start_scatter(1)
        wait_scatter(0)

        @pl.when(not_last)
        def _():
            load_dst(ka + 2, 0)
            wait_src_ew(0)
            start_gather(0)
        wait_scatter(1)

        @pl.when(not_last)
        def _():
            load_dst(ka + 3, 1)
        return 0
    lax.fori_loop(0, CPS // 2, pipe_body, 0)
    plsc.subcore_barrier()

    def relu_rows(buf, nrows):
        def relu_row(r, _):
            for j in range(HALF // 16):
                v = buf[r, pl.ds(j * 16, 16)]
                buf[r, pl.ds(j * 16, 16)] = jnp.maximum(v, 0.0)
            return 0
        lax.fori_loop(0, nrows, relu_row, 0)

    def wb_chunk(r0, nrows):
        pltpu.sync_copy(acc_sh.at[pl.ds(r0, nrows)],
                        rows0.at[pl.ds(0, nrows)])
        relu_rows(rows0, nrows)
        pltpu.sync_copy(rows0.at[pl.ds(0, nrows)],
                        out_hbm.at[pl.ds(r0, nrows), pl.ds(c * HALF, HALF)])

    for k in range(ROWS_T // ECHK):
        wb_chunk(s * ROWS_T + k * ECHK, ECHK)
    wb_chunk(s * ROWS_T + (ROWS_T // ECHK) * ECHK,
             ROWS_T - (ROWS_T // ECHK) * ECHK)

    @pl.when(s == NS - 1)
    def _():
        wb_chunk(NS * ROWS_T, TAIL)


@jax.jit
def _gnn(features, src, dst, ew, weight):
    sup = _matmul_halves(features, weight)
    mesh = plsc.VectorSubcoreMesh(core_axis_name="c", subcore_axis_name="s",
                                  num_cores=NC, num_subcores=NS)
    agg = pl.kernel(
        _sc_body,
        out_type=jax.ShapeDtypeStruct((N_NODES, F_OUT), jnp.float32),
        mesh=mesh,
        scratch_types=[
            pltpu.VMEM((ECHK,), jnp.int32),
            pltpu.VMEM((ECHK,), jnp.int32),
            pltpu.VMEM((ECHK,), jnp.int32),
            pltpu.VMEM((ECHK,), jnp.int32),
            pltpu.VMEM((ECHK + 16,), jnp.float32),
            pltpu.VMEM((ECHK + 16,), jnp.float32),
            pltpu.VMEM((ECHK, HALF), jnp.float32),
            pltpu.VMEM((ECHK, HALF), jnp.float32),
            pltpu.VMEM_SHARED((N_NODES, HALF), jnp.float32),
            pltpu.SemaphoreType.DMA,
            pltpu.SemaphoreType.DMA,
            pltpu.SemaphoreType.DMA,
            pltpu.SemaphoreType.DMA,
            pltpu.SemaphoreType.DMA,
            pltpu.SemaphoreType.DMA,
            pltpu.SemaphoreType.DMA,
            pltpu.SemaphoreType.DMA,
        ],
    )
    return agg(sup, src, dst, ew)


def kernel(features, edge_index, edge_weight, weight):
    pad = E_PAD - N_EDGES
    src = jnp.pad(edge_index[1].astype(jnp.int32), (0, pad))
    dst = jnp.pad(edge_index[0].astype(jnp.int32), (0, pad))
    ew = jnp.pad(edge_weight, (0, pad))
    return _gnn(features, src, dst, ew, weight)
